# baseline (speedup 1.0000x reference)
import functools

import jax
import jax.numpy as jnp
from jax import lax
from jax.experimental import pallas as pl
from jax.experimental.pallas import tpu as pltpu
from jax.experimental.pallas import tpu_sc as plsc

_CHUNK = 128


def _make_gather(num_rows: int, dim: int, batch: int):
    info = plsc.get_sparse_core_info()
    nw = info.num_cores * info.num_subcores
    nc = info.num_cores
    b_per_w = batch // nw
    chunks = b_per_w // _CHUNK

    mesh = plsc.VectorSubcoreMesh(core_axis_name="c", subcore_axis_name="s")

    @functools.partial(
        pl.kernel,
        mesh=mesh,
        out_type=jax.ShapeDtypeStruct((batch, dim), jnp.float32),
        compiler_params=pltpu.CompilerParams(use_tc_tiling_on_sc=False),
        scratch_types=[
            pltpu.VMEM((chunks, _CHUNK), jnp.int32),
            pltpu.VMEM((b_per_w, dim), jnp.float32),
            pltpu.SemaphoreType.DMA,
        ],
    )
    def gather(idx_hbm, table_hbm, out_hbm, idx_v, rows_v, sem):
        wid = lax.axis_index("s") * nc + lax.axis_index("c")
        base = wid * b_per_w
        pltpu.sync_copy(idx_hbm.at[pl.ds(wid * chunks, chunks)], idx_v)
        copies = [
            pltpu.async_copy(
                table_hbm.at[idx_v.at[j]],
                rows_v.at[pl.ds(j * _CHUNK, _CHUNK)],
                sem,
            )
            for j in range(chunks)
        ]
        for c in copies:
            c.wait()
        pltpu.sync_copy(rows_v, out_hbm.at[pl.ds(base, b_per_w)])

    return gather


def kernel(ind, z):
    batch = ind.shape[0]
    num_rows, dim = z.shape
    idx2d = ind.reshape(batch // _CHUNK, _CHUNK)
    return _make_gather(num_rows, dim, batch)(idx2d, z)


# speedup vs baseline: 1.6955x; 1.6955x over previous
import functools

import jax
import jax.numpy as jnp
from jax import lax
from jax.experimental import pallas as pl
from jax.experimental.pallas import tpu as pltpu
from jax.experimental.pallas import tpu_sc as plsc

_L = 16


def _make_gather(num_rows: int, dim: int, batch: int):
    info = plsc.get_sparse_core_info()
    nw = info.num_cores * info.num_subcores
    nc = info.num_cores
    b_per_w = batch // nw
    n_groups = b_per_w // _L

    mesh = plsc.VectorSubcoreMesh(core_axis_name="c", subcore_axis_name="s")

    @functools.partial(
        pl.kernel,
        mesh=mesh,
        out_type=jax.ShapeDtypeStruct((batch, dim), jnp.float32),
        compiler_params=pltpu.CompilerParams(needs_layout_passes=False),
        scratch_types=[
            pltpu.VMEM((b_per_w,), jnp.int32),
            pltpu.VMEM((b_per_w, dim), jnp.float32),
            pltpu.SemaphoreType.DMA,
        ],
    )
    def gather(idx_hbm, tab_hbm, out_hbm, idx_v, out_v, sem):
        wid = lax.axis_index("s") * nc + lax.axis_index("c")
        base = wid * b_per_w
        pltpu.sync_copy(idx_hbm.at[pl.ds(base, b_per_w)], idx_v)

        def group_body(g, carry):
            for l in range(_L):
                j = g * _L + l
                s = jnp.max(plsc.load_gather(idx_v, [jnp.broadcast_to(j, (_L,))]))
                pltpu.async_copy(
                    tab_hbm.at[pl.ds(s, 1)], out_v.at[pl.ds(j, 1)], sem
                )
            return carry

        lax.fori_loop(0, n_groups, group_body, 0)

        pltpu.make_async_copy(tab_hbm.at[pl.ds(0, b_per_w)], out_v, sem).wait()

        pltpu.sync_copy(out_v, out_hbm.at[pl.ds(base, b_per_w)])

    return gather


def kernel(ind, z):
    batch = ind.shape[0]
    num_rows, dim = z.shape
    return _make_gather(num_rows, dim, batch)(ind, z)


# speedup vs baseline: 1.6998x; 1.0025x over previous
import functools

import jax
import jax.numpy as jnp
from jax import lax
from jax.experimental import pallas as pl
from jax.experimental.pallas import tpu as pltpu
from jax.experimental.pallas import tpu_sc as plsc

_L = 16


def _make_gather(num_rows: int, dim: int, batch: int):
    info = plsc.get_sparse_core_info()
    nw = info.num_cores * info.num_subcores
    nc = info.num_cores
    b_per_w = batch // nw
    n_groups = b_per_w // _L

    mesh = plsc.VectorSubcoreMesh(core_axis_name="c", subcore_axis_name="s")

    @functools.partial(
        pl.kernel,
        mesh=mesh,
        out_type=jax.ShapeDtypeStruct((batch, dim), jnp.float32),
        compiler_params=pltpu.CompilerParams(
            needs_layout_passes=False,
            skip_device_barrier=True,
            disable_bounds_checks=True,
            disable_semaphore_checks=True,
        ),
        scratch_types=[
            pltpu.VMEM((b_per_w,), jnp.int32),
            pltpu.VMEM((b_per_w, dim), jnp.float32),
            pltpu.SemaphoreType.DMA,
        ],
    )
    def gather(idx_hbm, tab_hbm, out_hbm, idx_v, out_v, sem):
        wid = lax.axis_index("s") * nc + lax.axis_index("c")
        base = wid * b_per_w
        pltpu.sync_copy(idx_hbm.at[pl.ds(base, b_per_w)], idx_v)

        def group_body(g, carry):
            for l in range(_L):
                j = g * _L + l
                s = jnp.max(plsc.load_gather(idx_v, [jnp.broadcast_to(j, (_L,))]))
                pltpu.async_copy(
                    tab_hbm.at[pl.ds(s, 1)], out_v.at[pl.ds(j, 1)], sem
                )
            return carry

        lax.fori_loop(0, n_groups, group_body, 0)

        pltpu.make_async_copy(tab_hbm.at[pl.ds(0, b_per_w)], out_v, sem).wait()

        pltpu.sync_copy(out_v, out_hbm.at[pl.ds(base, b_per_w)])

    return gather


def kernel(ind, z):
    batch = ind.shape[0]
    num_rows, dim = z.shape
    return _make_gather(num_rows, dim, batch)(ind, z)


# speedup vs baseline: 3.8915x; 2.2894x over previous
import functools

import jax
import jax.numpy as jnp
from jax import lax
from jax.experimental import pallas as pl
from jax.experimental.pallas import tpu as pltpu
from jax.experimental.pallas import tpu_sc as plsc

_L = 16
_CW = 512


def _make_gather(num_rows: int, dim: int, batch: int):
    info = plsc.get_sparse_core_info()
    nc, ns = info.num_cores, info.num_subcores
    nw = nc * ns
    nblk_all = num_rows // 128
    tail0 = nblk_all * 128
    n_tail = num_rows - tail0
    base_blk = nblk_all // nw
    extra = nblk_all - base_blk * nw
    share = batch // ns

    mesh = plsc.VectorSubcoreMesh(core_axis_name="c", subcore_axis_name="s")

    @functools.partial(
        pl.kernel,
        mesh=mesh,
        out_type=jax.ShapeDtypeStruct((batch * dim,), jnp.float32),
        compiler_params=pltpu.CompilerParams(
            needs_layout_passes=False,
            skip_device_barrier=True,
            disable_bounds_checks=True,
            disable_semaphore_checks=True,
        ),
        scratch_types=[
            pltpu.VMEM((batch,), jnp.int32),
            pltpu.VMEM((batch,), jnp.int32),
            pltpu.VMEM((batch,), jnp.int32),
            pltpu.VMEM((dim, _CW), jnp.float32),
            pltpu.VMEM((dim, _CW), jnp.float32),
            pltpu.VMEM((dim,), jnp.float32),
            pltpu.VMEM((n_tail, dim), jnp.float32),
            pltpu.SemaphoreType.DMA,
            pltpu.SemaphoreType.DMA,
        ],
    )
    def gather(
        idx_hbm, tab_hbm, tail_hbm, out_hbm, idx_v, kept_v, work_v,
        buf0, buf1, rowb, tailv, sem0, sem1,
    ):
        cid = lax.axis_index("c")
        sid = lax.axis_index("s")
        w = sid * nc + cid
        lane = lax.iota(jnp.int32, _L)

        start_blk = w * base_blk + jnp.minimum(w, extra)
        n_blk = base_blk + jnp.where(w < extra, 1, 0)
        col_lo = start_blk * 128
        col_hi = (start_blk + n_blk) * 128
        col_hi = jnp.where(w == nw - 1, num_rows, col_hi)
        n_ch = (n_blk * 128 + _CW - 1) // _CW

        pltpu.sync_copy(idx_hbm, idx_v)

        def chunk_col0(k):
            return col_lo + k * _CW

        pltpu.async_copy(
            tab_hbm.at[:, pl.ds(chunk_col0(0), _CW)], buf0, sem0
        )

        def scan_body(g, p):
            iv = lane + g * _L
            rv = idx_v[pl.ds(g * _L, _L)]
            m = (rv >= col_lo) & (rv < col_hi)
            plsc.store_compressed(kept_v.at[pl.ds(p, _L)], iv, mask=m)
            return p + jnp.max(plsc.all_reduce_population_count(m))

        n_kept = lax.fori_loop(0, batch // _L, scan_body, jnp.int32(0))

        @pl.when(1 < n_ch)
        def _():
            pltpu.async_copy(
                tab_hbm.at[:, pl.ds(chunk_col0(1), _CW)], buf1, sem1
            )

        def select(buf, c0, cwidth):
            def wscan(t, q):
                valid = (lane + t * _L) < n_kept
                kv = plsc.load_gather(kept_v, [lane + t * _L], mask=valid)
                rv = plsc.load_gather(idx_v, [kv], mask=valid)
                m = valid & (rv >= c0) & (rv < c0 + cwidth)
                plsc.store_compressed(work_v.at[pl.ds(q, _L)], kv, mask=m)
                return q + jnp.max(plsc.all_reduce_population_count(m))

            n_work = lax.fori_loop(
                0, (n_kept + _L - 1) // _L, wscan, jnp.int32(0)
            )

            def emit(e, carry):
                isplat = plsc.load_gather(work_v, [jnp.broadcast_to(e, (_L,))])
                rsplat = plsc.load_gather(idx_v, [isplat])
                colv = rsplat - c0
                for f0 in range(0, dim, _L):
                    rowb[pl.ds(f0, _L)] = plsc.load_gather(
                        buf, [lane + f0, colv]
                    )
                i_s = jnp.max(isplat)
                pltpu.sync_copy(rowb, out_hbm.at[pl.ds(i_s * dim, dim)])
                return carry

            lax.fori_loop(0, n_work, emit, 0)

        def pair_body(q, carry):
            for b, (buf, sem) in enumerate(((buf0, sem0), (buf1, sem1))):
                k = q * 2 + b

                @pl.when(k < n_ch)
                def _():
                    pltpu.make_async_copy(
                        tab_hbm.at[:, pl.ds(0, _CW)], buf, sem
                    ).wait()
                    select(buf, chunk_col0(k), _CW)

                    @pl.when(k + 2 < n_ch)
                    def _():
                        pltpu.async_copy(
                            tab_hbm.at[:, pl.ds(chunk_col0(k + 2), _CW)],
                            buf,
                            sem,
                        )

            return carry

        max_pairs = (base_blk * 128 // _CW + 1 + 1) // 2 + 1
        lax.fori_loop(0, max_pairs, pair_body, 0)

        @pl.when(w == nw - 1)
        def _():
            pltpu.sync_copy(tail_hbm, tailv)

            def twscan(t, q):
                valid = (lane + t * _L) < n_kept
                kv = plsc.load_gather(kept_v, [lane + t * _L], mask=valid)
                rv = plsc.load_gather(idx_v, [kv], mask=valid)
                m = valid & (rv >= tail0)
                plsc.store_compressed(work_v.at[pl.ds(q, _L)], kv, mask=m)
                return q + jnp.max(plsc.all_reduce_population_count(m))

            n_tw = lax.fori_loop(
                0, (n_kept + _L - 1) // _L, twscan, jnp.int32(0)
            )

            def temit(e, carry):
                isplat = plsc.load_gather(work_v, [jnp.broadcast_to(e, (_L,))])
                rsplat = plsc.load_gather(idx_v, [isplat])
                rloc = rsplat - tail0
                for f0 in range(0, dim, _L):
                    rowb[pl.ds(f0, _L)] = plsc.load_gather(
                        tailv, [rloc, lane + f0]
                    )
                i_s = jnp.max(isplat)
                pltpu.sync_copy(rowb, out_hbm.at[pl.ds(i_s * dim, dim)])
                return carry

            lax.fori_loop(0, n_tw, temit, 0)


    return gather


def kernel(ind, z):
    batch = ind.shape[0]
    num_rows, dim = z.shape
    tail0 = (num_rows // 128) * 128
    flat = _make_gather(num_rows, dim, batch)(ind, z.T, z[tail0:])
    return flat.reshape(batch, dim)
